# baseline (speedup 1.0000x reference)
import functools
import math

import jax
import jax.numpy as jnp
from jax import lax
from jax.experimental import pallas as pl
from jax.experimental.pallas import tpu as pltpu
from jax.experimental.pallas import tpu_sc as plsc

D_MODEL = 64
D_PAD = 128
SCALE = math.sqrt(D_MODEL)
NBUF = 3


@functools.partial(jax.jit, static_argnames=("rows", "cols"))
def _embed(tokens, W_padded, *, rows, cols):
  info = plsc.get_sparse_core_info()
  nc, ns, nl = info.num_cores, info.num_subcores, info.num_lanes
  nw = nc * ns
  rows_per_w = rows // nw
  ngroup = cols // NBUF
  d = D_MODEL

  mesh = plsc.VectorSubcoreMesh(core_axis_name="c", subcore_axis_name="s")

  @functools.partial(
      pl.kernel,
      out_type=jax.ShapeDtypeStruct((cols, d, rows), jnp.float32),
      mesh=mesh,
      compiler_params=pltpu.CompilerParams(needs_layout_passes=False),
      scratch_types=(
          [pltpu.VMEM((rows_per_w, cols), jnp.int32)]
          + [pltpu.VMEM((rows_per_w,), jnp.int32) for _ in range(NBUF)]
          + [pltpu.VMEM((rows_per_w, D_PAD), jnp.float32) for _ in range(NBUF)]
          + [pltpu.VMEM((d, rows_per_w), jnp.float32) for _ in range(NBUF)]
          + [pltpu.SemaphoreType.DMA for _ in range(2 * NBUF)]
      ),
  )
  def k(tok_hbm, w_hbm, out_hbm, tv, *bufs_and_sems):
    cidx = bufs_and_sems[:NBUF]
    in_bufs = bufs_and_sems[NBUF:2 * NBUF]
    out_bufs = bufs_and_sems[2 * NBUF:3 * NBUF]
    g_sems = bufs_and_sems[3 * NBUF:4 * NBUF]
    s_sems = bufs_and_sems[4 * NBUF:]

    wid = lax.axis_index("s") * nc + lax.axis_index("c")
    i0 = wid * rows_per_w

    pltpu.sync_copy(tok_hbm.at[pl.ds(i0, rows_per_w), :], tv)

    lane_iota = lax.iota(jnp.int32, nl)

    def fire_gather(j, b):
      jv = jnp.full((nl,), j, jnp.int32)
      for blk in range(rows_per_w // nl):
        v = plsc.load_gather(tv, [lane_iota + blk * nl, jv])
        cidx[b][pl.ds(blk * nl, nl)] = v
      pltpu.async_copy(w_hbm.at[cidx[b]], in_bufs[b], g_sems[b])

    def wait_gather(b):
      pltpu.make_async_copy(
          w_hbm.at[pl.ds(0, rows_per_w)], in_bufs[b], g_sems[b]).wait()

    def fire_scatter(j, b):
      pltpu.async_copy(
          out_bufs[b], out_hbm.at[j, :, pl.ds(i0, rows_per_w)], s_sems[b])

    def wait_scatter(b):
      pltpu.make_async_copy(
          out_bufs[b], out_hbm.at[0, :, pl.ds(i0, rows_per_w)], s_sems[b]).wait()

    def transpose_scale(b):
      @plsc.parallel_loop(0, rows_per_w, unroll=2)
      def _t(i):
        iv = jnp.full((nl,), i, jnp.int32)
        for kb in range(d // nl):
          v = in_bufs[b][i, pl.ds(kb * nl, nl)] * SCALE
          plsc.store_scatter(out_bufs[b], [lane_iota + kb * nl, iv], v)

    for b in range(NBUF):
      fire_gather(b, b)

    @pl.loop(0, ngroup)
    def _group(g):
      for b in range(NBUF):
        j = g * NBUF + b
        wait_gather(b)

        @pl.when(j >= NBUF)
        def _():
          wait_scatter(b)

        transpose_scale(b)
        fire_scatter(j, b)

        @pl.when(j + NBUF < cols)
        def _():
          fire_gather(j + NBUF, b)

    for r in range(cols - ngroup * NBUF):
      j = ngroup * NBUF + r
      wait_gather(r)
      wait_scatter(r)
      transpose_scale(r)
      fire_scatter(j, r)

    for b in range(NBUF):
      wait_scatter(b)

  return k(tokens, W_padded)


def kernel(tokens, W):
  rows, cols = tokens.shape
  tokens = tokens.astype(jnp.int32)
  W_padded = jnp.pad(W, ((0, 0), (0, D_PAD - D_MODEL)))
  out_t = _embed(tokens, W_padded, rows=rows, cols=cols)
  return jnp.transpose(out_t, (2, 0, 1))


# speedup vs baseline: 2.0761x; 2.0761x over previous
import functools
import math

import jax
import jax.numpy as jnp
from jax import lax
from jax.experimental import pallas as pl
from jax.experimental.pallas import tpu as pltpu
from jax.experimental.pallas import tpu_sc as plsc

D_MODEL = 64
D_PAD = 128
SCALE = math.sqrt(D_MODEL)
NBUF = 3


@functools.partial(jax.jit, static_argnames=("rows", "cols"))
def _embed(tokens, W_padded, *, rows, cols):
  info = plsc.get_sparse_core_info()
  nc, ns, nl = info.num_cores, info.num_subcores, info.num_lanes
  nw = nc * ns
  rows_per_w = rows // nw
  ngroup = cols // NBUF
  d = D_MODEL

  mesh = plsc.VectorSubcoreMesh(core_axis_name="c", subcore_axis_name="s")

  @functools.partial(
      pl.kernel,
      out_type=jax.ShapeDtypeStruct((cols, d, rows), jnp.float32),
      mesh=mesh,
      compiler_params=pltpu.CompilerParams(needs_layout_passes=False),
      scratch_types=(
          [pltpu.VMEM((rows_per_w, cols), jnp.int32)]
          + [pltpu.VMEM((rows_per_w,), jnp.int32) for _ in range(NBUF)]
          + [pltpu.VMEM((rows_per_w, D_PAD), jnp.float32) for _ in range(NBUF)]
          + [pltpu.VMEM((d, rows_per_w), jnp.float32) for _ in range(NBUF)]
          + [pltpu.SemaphoreType.DMA for _ in range(2 * NBUF)]
      ),
  )
  def k(tok_hbm, w_hbm, out_hbm, tv, *bufs_and_sems):
    cidx = bufs_and_sems[:NBUF]
    in_bufs = bufs_and_sems[NBUF:2 * NBUF]
    out_bufs = bufs_and_sems[2 * NBUF:3 * NBUF]
    g_sems = bufs_and_sems[3 * NBUF:4 * NBUF]
    s_sems = bufs_and_sems[4 * NBUF:]

    wid = lax.axis_index("s") * nc + lax.axis_index("c")
    i0 = wid * rows_per_w

    pltpu.sync_copy(tok_hbm.at[pl.ds(i0, rows_per_w), :], tv)

    lane_iota = lax.iota(jnp.int32, nl)

    def fire_gather(j, b):
      jv = jnp.full((nl,), j, jnp.int32)
      for blk in range(rows_per_w // nl):
        v = plsc.load_gather(tv, [lane_iota + blk * nl, jv])
        cidx[b][pl.ds(blk * nl, nl)] = v
      pltpu.async_copy(w_hbm.at[cidx[b]], in_bufs[b], g_sems[b])

    def wait_gather(b):
      pltpu.make_async_copy(
          w_hbm.at[pl.ds(0, rows_per_w)], in_bufs[b], g_sems[b]).wait()

    def fire_scatter(j, b):
      pltpu.async_copy(
          out_bufs[b], out_hbm.at[j, :, pl.ds(i0, rows_per_w)], s_sems[b])

    def wait_scatter(b):
      pltpu.make_async_copy(
          out_bufs[b], out_hbm.at[0, :, pl.ds(i0, rows_per_w)], s_sems[b]).wait()

    def transpose_scale(b):
      @plsc.parallel_loop(0, nl, unroll=2)
      def _t(s):
        rot = lax.bitwise_and(lane_iota + s, nl - 1)
        for ib in range(rows_per_w // nl):
          iv = rot + ib * nl
          for kb in range(d // nl):
            kv = lane_iota + kb * nl
            v = plsc.load_gather(in_bufs[b], [iv, kv]) * SCALE
            plsc.store_scatter(out_bufs[b], [kv, iv], v)

    for b in range(NBUF):
      fire_gather(b, b)

    @pl.loop(0, ngroup)
    def _group(g):
      for b in range(NBUF):
        j = g * NBUF + b
        wait_gather(b)

        @pl.when(j >= NBUF)
        def _():
          wait_scatter(b)

        transpose_scale(b)
        fire_scatter(j, b)

        @pl.when(j + NBUF < cols)
        def _():
          fire_gather(j + NBUF, b)

    for r in range(cols - ngroup * NBUF):
      j = ngroup * NBUF + r
      wait_gather(r)
      wait_scatter(r)
      transpose_scale(r)
      fire_scatter(j, r)

    for b in range(NBUF):
      wait_scatter(b)

  return k(tokens, W_padded)


def kernel(tokens, W):
  rows, cols = tokens.shape
  tokens = tokens.astype(jnp.int32)
  W_padded = jnp.pad(W, ((0, 0), (0, D_PAD - D_MODEL)))
  out_t = _embed(tokens, W_padded, rows=rows, cols=cols)
  return jnp.transpose(out_t, (2, 0, 1))


# speedup vs baseline: 2.1474x; 1.0343x over previous
import functools
import math

import jax
import jax.numpy as jnp
from jax import lax
from jax.experimental import pallas as pl
from jax.experimental.pallas import tpu as pltpu
from jax.experimental.pallas import tpu_sc as plsc

D_MODEL = 64
D_PAD = 128
SCALE = math.sqrt(D_MODEL)
NBUF = 4


@functools.partial(jax.jit, static_argnames=("rows", "cols"))
def _embed(tokens_flat, W_padded, *, rows, cols):
  info = plsc.get_sparse_core_info()
  nc, ns, nl = info.num_cores, info.num_subcores, info.num_lanes
  nw = nc * ns
  rows_per_w = rows // nw
  b_per_w = rows_per_w * cols
  ngroup = cols // NBUF
  d = D_MODEL

  mesh = plsc.VectorSubcoreMesh(core_axis_name="c", subcore_axis_name="s")

  @functools.partial(
      pl.kernel,
      out_type=jax.ShapeDtypeStruct((cols, d, rows), jnp.float32),
      mesh=mesh,
      compiler_params=pltpu.CompilerParams(needs_layout_passes=False),
      scratch_types=(
          [pltpu.VMEM((b_per_w,), jnp.int32)]
          + [pltpu.VMEM((rows_per_w,), jnp.int32) for _ in range(NBUF)]
          + [pltpu.VMEM((rows_per_w, D_PAD), jnp.float32) for _ in range(NBUF)]
          + [pltpu.VMEM((d, rows_per_w), jnp.float32) for _ in range(NBUF)]
          + [pltpu.SemaphoreType.DMA for _ in range(2 * NBUF)]
      ),
  )
  def k(tok_hbm, w_hbm, out_hbm, tv, *bufs_and_sems):
    cidx = bufs_and_sems[:NBUF]
    in_bufs = bufs_and_sems[NBUF:2 * NBUF]
    out_bufs = bufs_and_sems[2 * NBUF:3 * NBUF]
    g_sems = bufs_and_sems[3 * NBUF:4 * NBUF]
    s_sems = bufs_and_sems[4 * NBUF:]

    wid = lax.axis_index("s") * nc + lax.axis_index("c")
    i0 = wid * rows_per_w

    pltpu.sync_copy(tok_hbm.at[pl.ds(wid * b_per_w, b_per_w)], tv)

    lane_iota = lax.iota(jnp.int32, nl)

    def fire_gather(j, b):
      for blk in range(rows_per_w // nl):
        v = plsc.load_gather(tv, [(lane_iota + blk * nl) * cols + j])
        cidx[b][pl.ds(blk * nl, nl)] = v
      pltpu.async_copy(w_hbm.at[cidx[b]], in_bufs[b], g_sems[b])

    def wait_gather(b):
      pltpu.make_async_copy(
          w_hbm.at[pl.ds(0, rows_per_w)], in_bufs[b], g_sems[b]).wait()

    def fire_scatter(j, b):
      pltpu.async_copy(
          out_bufs[b], out_hbm.at[j, :, pl.ds(i0, rows_per_w)], s_sems[b])

    def wait_scatter(b):
      pltpu.make_async_copy(
          out_bufs[b], out_hbm.at[0, :, pl.ds(i0, rows_per_w)], s_sems[b]).wait()

    def transpose_scale(b):
      @plsc.parallel_loop(0, nl, unroll=2)
      def _t(s):
        rot = lax.bitwise_and(lane_iota + s, nl - 1)
        for ib in range(rows_per_w // nl):
          iv = rot + ib * nl
          for kb in range(d // nl):
            kv = lane_iota + kb * nl
            v = plsc.load_gather(in_bufs[b], [iv, kv]) * SCALE
            plsc.store_scatter(out_bufs[b], [kv, iv], v)

    for b in range(NBUF):
      fire_gather(b, b)

    @pl.loop(0, ngroup)
    def _group(g):
      for b in range(NBUF):
        j = g * NBUF + b
        wait_gather(b)

        @pl.when(j >= NBUF)
        def _():
          wait_scatter(b)

        transpose_scale(b)
        fire_scatter(j, b)

        @pl.when(j + NBUF < cols)
        def _():
          fire_gather(j + NBUF, b)

    for r in range(cols - ngroup * NBUF):
      j = ngroup * NBUF + r
      wait_gather(r)
      wait_scatter(r)
      transpose_scale(r)
      fire_scatter(j, r)

    for b in range(NBUF):
      wait_scatter(b)

  return k(tokens_flat, W_padded)


def kernel(tokens, W):
  rows, cols = tokens.shape
  tokens_flat = tokens.astype(jnp.int32).reshape(-1)
  W_padded = jnp.pad(W, ((0, 0), (0, D_PAD - D_MODEL)))
  out_t = _embed(tokens_flat, W_padded, rows=rows, cols=cols)
  return jnp.transpose(out_t, (2, 0, 1))


# speedup vs baseline: 2.1795x; 1.0149x over previous
import functools
import math

import jax
import jax.numpy as jnp
from jax import lax
from jax.experimental import pallas as pl
from jax.experimental.pallas import tpu as pltpu
from jax.experimental.pallas import tpu_sc as plsc

D_MODEL = 64
D_PAD = 128
SCALE = math.sqrt(D_MODEL)
NBUF = 4


@functools.partial(jax.jit, static_argnames=("rows", "cols"))
def _embed(tokens_flat, W_padded, *, rows, cols):
  info = plsc.get_sparse_core_info()
  nc, ns, nl = info.num_cores, info.num_subcores, info.num_lanes
  nw = nc * ns
  rows_per_w = rows // nw
  b_per_w = rows_per_w * cols
  ngroup = cols // NBUF
  d = D_MODEL

  mesh = plsc.VectorSubcoreMesh(core_axis_name="c", subcore_axis_name="s")

  @functools.partial(
      pl.kernel,
      out_type=jax.ShapeDtypeStruct((cols, d, rows), jnp.float32),
      mesh=mesh,
      compiler_params=pltpu.CompilerParams(needs_layout_passes=False),
      scratch_types=(
          [pltpu.VMEM((b_per_w,), jnp.int32)]
          + [pltpu.VMEM((rows_per_w,), jnp.int32) for _ in range(NBUF)]
          + [pltpu.VMEM((rows_per_w, D_PAD), jnp.float32) for _ in range(NBUF)]
          + [pltpu.VMEM((d, rows_per_w), jnp.float32) for _ in range(NBUF)]
          + [pltpu.SemaphoreType.DMA for _ in range(2 * NBUF)]
      ),
  )
  def k(tok_hbm, w_hbm, out_hbm, tv, *bufs_and_sems):
    cidx = bufs_and_sems[:NBUF]
    in_bufs = bufs_and_sems[NBUF:2 * NBUF]
    out_bufs = bufs_and_sems[2 * NBUF:3 * NBUF]
    g_sems = bufs_and_sems[3 * NBUF:4 * NBUF]
    s_sems = bufs_and_sems[4 * NBUF:]

    wid = lax.axis_index("s") * nc + lax.axis_index("c")
    i0 = wid * rows_per_w

    pltpu.sync_copy(tok_hbm.at[pl.ds(wid * b_per_w, b_per_w)], tv)

    lane_iota = lax.iota(jnp.int32, nl)

    def fire_gather(j, b):
      for blk in range(rows_per_w // nl):
        v = plsc.load_gather(tv, [(lane_iota + blk * nl) * cols + j])
        cidx[b][pl.ds(blk * nl, nl)] = v
      pltpu.async_copy(w_hbm.at[cidx[b]], in_bufs[b], g_sems[b])

    def wait_gather(b):
      pltpu.make_async_copy(
          w_hbm.at[pl.ds(0, rows_per_w)], in_bufs[b], g_sems[b]).wait()

    def fire_scatter(j, b):
      pltpu.async_copy(
          out_bufs[b], out_hbm.at[j, :, pl.ds(i0, rows_per_w)], s_sems[b])

    def wait_scatter(b):
      pltpu.make_async_copy(
          out_bufs[b], out_hbm.at[0, :, pl.ds(i0, rows_per_w)], s_sems[b]).wait()

    def transpose_scale(b):
      @plsc.parallel_loop(0, nl, unroll=4)
      def _t(s):
        rot = lax.bitwise_and(lane_iota + s, nl - 1)
        for ib in range(rows_per_w // nl):
          iv = rot + ib * nl
          for kb in range(d // nl):
            kv = lane_iota + kb * nl
            v = plsc.load_gather(in_bufs[b], [iv, kv])
            plsc.store_scatter(out_bufs[b], [kv, iv], v)

    for b in range(NBUF):
      fire_gather(b, b)

    @pl.loop(0, ngroup)
    def _group(g):
      for b in range(NBUF):
        j = g * NBUF + b
        wait_gather(b)

        @pl.when(j >= NBUF)
        def _():
          wait_scatter(b)

        transpose_scale(b)
        fire_scatter(j, b)

        @pl.when(j + NBUF < cols)
        def _():
          fire_gather(j + NBUF, b)

    for r in range(cols - ngroup * NBUF):
      j = ngroup * NBUF + r
      wait_gather(r)
      wait_scatter(r)
      transpose_scale(r)
      fire_scatter(j, r)

    for b in range(NBUF):
      wait_scatter(b)

  return k(tokens_flat, W_padded)


def kernel(tokens, W):
  rows, cols = tokens.shape
  tokens_flat = tokens.astype(jnp.int32).reshape(-1)
  W_padded = jnp.pad(W * SCALE, ((0, 0), (0, D_PAD - D_MODEL)))
  out_t = _embed(tokens_flat, W_padded, rows=rows, cols=cols)
  return jnp.transpose(out_t, (2, 0, 1))


# speedup vs baseline: 2.2830x; 1.0475x over previous
import functools
import math

import jax
import jax.numpy as jnp
from jax import lax
from jax.experimental import pallas as pl
from jax.experimental.pallas import tpu as pltpu
from jax.experimental.pallas import tpu_sc as plsc

D_MODEL = 64
D_PAD = 128
SCALE = math.sqrt(D_MODEL)
NBUF = 4


@functools.partial(jax.jit, static_argnames=("rows", "cols"))
def _embed(tokens_flat, W_padded, *, rows, cols):
  info = plsc.get_sparse_core_info()
  nc, ns, nl = info.num_cores, info.num_subcores, info.num_lanes
  nw = nc * ns
  rows_per_w = rows // nw
  b_per_w = rows_per_w * cols
  ngroup = cols // NBUF
  d = D_MODEL

  mesh = plsc.VectorSubcoreMesh(core_axis_name="c", subcore_axis_name="s")

  @functools.partial(
      pl.kernel,
      out_type=jax.ShapeDtypeStruct((cols, d, rows), jnp.float32),
      mesh=mesh,
      compiler_params=pltpu.CompilerParams(needs_layout_passes=False),
      scratch_types=(
          [pltpu.VMEM((b_per_w,), jnp.int32)]
          + [pltpu.VMEM((rows_per_w,), jnp.int32) for _ in range(NBUF)]
          + [pltpu.VMEM((rows_per_w, D_PAD), jnp.float32) for _ in range(NBUF)]
          + [pltpu.VMEM((d, rows_per_w), jnp.float32) for _ in range(NBUF)]
          + [pltpu.SemaphoreType.DMA for _ in range(2 * NBUF)]
      ),
  )
  def k(tok_hbm, w_hbm, out_hbm, tv, *bufs_and_sems):
    cidx = bufs_and_sems[:NBUF]
    in_bufs = bufs_and_sems[NBUF:2 * NBUF]
    out_bufs = bufs_and_sems[2 * NBUF:3 * NBUF]
    g_sems = bufs_and_sems[3 * NBUF:4 * NBUF]
    s_sems = bufs_and_sems[4 * NBUF:]

    wid = lax.axis_index("s") * nc + lax.axis_index("c")
    i0 = wid * rows_per_w

    pltpu.sync_copy(tok_hbm.at[pl.ds(wid * b_per_w, b_per_w)], tv)

    lane_iota = lax.iota(jnp.int32, nl)

    half_rows = rows_per_w // 2

    def fire_gather_half(j, b, half):
      for blk in range(half * 4, half * 4 + 4):
        v = plsc.load_gather(tv, [(lane_iota + blk * nl) * cols + j])
        cidx[b][pl.ds(blk * nl, nl)] = v
      pltpu.async_copy(
          w_hbm.at[cidx[b].at[pl.ds(half * half_rows, half_rows)]],
          in_bufs[b].at[pl.ds(half * half_rows, half_rows)], g_sems[b])

    def fire_gather(j, b):
      fire_gather_half(j, b, 0)
      fire_gather_half(j, b, 1)

    def wait_gather(b):
      pltpu.make_async_copy(
          w_hbm.at[pl.ds(0, rows_per_w)], in_bufs[b], g_sems[b]).wait()

    half_d = d // 2

    def fire_scatter_half(j, b, half):
      pltpu.async_copy(
          out_bufs[b].at[pl.ds(half * half_d, half_d), :],
          out_hbm.at[j, pl.ds(half * half_d, half_d), pl.ds(i0, rows_per_w)],
          s_sems[b])

    def fire_scatter(j, b):
      fire_scatter_half(j, b, 0)
      fire_scatter_half(j, b, 1)

    def wait_scatter(b):
      pltpu.make_async_copy(
          out_bufs[b], out_hbm.at[0, :, pl.ds(i0, rows_per_w)], s_sems[b]).wait()

    def transpose_half(b, half):
      @plsc.parallel_loop(0, nl, unroll=4)
      def _t(s):
        rot = lax.bitwise_and(lane_iota + s, nl - 1)
        for ib in range(rows_per_w // nl):
          iv = rot + ib * nl
          for kb in range(half * 2, half * 2 + 2):
            kv = lane_iota + kb * nl
            v = plsc.load_gather(in_bufs[b], [iv, kv])
            plsc.store_scatter(out_bufs[b], [kv, iv], v)

    def transpose_scale(b):
      transpose_half(b, 0)
      transpose_half(b, 1)

    for b in range(NBUF):
      fire_gather(b, b)

    @pl.loop(0, ngroup)
    def _group(g):
      for b in range(NBUF):
        j = g * NBUF + b
        wait_gather(b)

        @pl.when(j >= NBUF)
        def _():
          wait_scatter(b)

        for half in (0, 1):
          transpose_half(b, half)
          fire_scatter_half(j, b, half)

        @pl.when(j + NBUF < cols)
        def _():
          fire_gather(j + NBUF, b)

    for r in range(cols - ngroup * NBUF):
      j = ngroup * NBUF + r
      wait_gather(r)
      wait_scatter(r)
      transpose_scale(r)
      fire_scatter(j, r)

    for b in range(NBUF):
      wait_scatter(b)

  return k(tokens_flat, W_padded)


def kernel(tokens, W):
  rows, cols = tokens.shape
  tokens_flat = tokens.astype(jnp.int32).reshape(-1)
  W_padded = jnp.pad(W * SCALE, ((0, 0), (0, D_PAD - D_MODEL)))
  out_t = _embed(tokens_flat, W_padded, rows=rows, cols=cols)
  return jnp.transpose(out_t, (2, 0, 1))


# speedup vs baseline: 2.4286x; 1.0638x over previous
import functools
import math

import jax
import jax.numpy as jnp
from jax import lax
from jax.experimental import pallas as pl
from jax.experimental.pallas import tpu as pltpu
from jax.experimental.pallas import tpu_sc as plsc

D_MODEL = 64
D_PAD = 128
SCALE = math.sqrt(D_MODEL)
NBUF = 4


@functools.partial(jax.jit, static_argnames=("rows", "cols"))
def _embed(tokens_flat, W_padded, *, rows, cols):
  info = plsc.get_sparse_core_info()
  nc, ns, nl = info.num_cores, info.num_subcores, info.num_lanes
  nw = nc * ns
  rows_per_w = rows // nw
  b_per_w = rows_per_w * cols
  ngroup = cols // NBUF
  d = D_MODEL

  mesh = plsc.VectorSubcoreMesh(core_axis_name="c", subcore_axis_name="s")

  @functools.partial(
      pl.kernel,
      out_type=jax.ShapeDtypeStruct((cols, d, rows), jnp.float32),
      mesh=mesh,
      compiler_params=pltpu.CompilerParams(needs_layout_passes=False),
      scratch_types=(
          [pltpu.VMEM((b_per_w,), jnp.int32)]
          + [pltpu.VMEM((rows_per_w,), jnp.int32) for _ in range(NBUF)]
          + [pltpu.VMEM((rows_per_w, D_PAD), jnp.float32) for _ in range(NBUF)]
          + [pltpu.VMEM((d, rows_per_w), jnp.float32) for _ in range(NBUF)]
          + [pltpu.SemaphoreType.DMA for _ in range(2 * NBUF)]
      ),
  )
  def k(tok_hbm, w_hbm, out_hbm, tv, *bufs_and_sems):
    cidx = bufs_and_sems[:NBUF]
    in_bufs = bufs_and_sems[NBUF:2 * NBUF]
    out_bufs = bufs_and_sems[2 * NBUF:3 * NBUF]
    g_sems = bufs_and_sems[3 * NBUF:4 * NBUF]
    s_sems = bufs_and_sems[4 * NBUF:]

    wid = lax.axis_index("s") * nc + lax.axis_index("c")
    i0 = wid * rows_per_w

    pltpu.sync_copy(tok_hbm.at[pl.ds(wid * b_per_w, b_per_w)], tv)

    lane_iota = lax.iota(jnp.int32, nl)

    half_rows = rows_per_w // 2

    def fire_gather_half(j, b, half):
      for blk in range(half * 4, half * 4 + 4):
        v = plsc.load_gather(tv, [(lane_iota + blk * nl) * cols + j])
        cidx[b][pl.ds(blk * nl, nl)] = v
      pltpu.async_copy(
          w_hbm.at[cidx[b].at[pl.ds(half * half_rows, half_rows)]],
          in_bufs[b].at[pl.ds(half * half_rows, half_rows)], g_sems[b])

    def fire_gather(j, b):
      fire_gather_half(j, b, 0)
      fire_gather_half(j, b, 1)

    def wait_gather(b):
      pltpu.make_async_copy(
          w_hbm.at[pl.ds(0, rows_per_w)], in_bufs[b], g_sems[b]).wait()

    half_d = d // 2

    def fire_scatter_half(j, b, half):
      pltpu.async_copy(
          out_bufs[b].at[pl.ds(half * half_d, half_d), :],
          out_hbm.at[j, pl.ds(half * half_d, half_d), pl.ds(i0, rows_per_w)],
          s_sems[b])

    def fire_scatter(j, b):
      fire_scatter_half(j, b, 0)
      fire_scatter_half(j, b, 1)

    def wait_scatter(b):
      pltpu.make_async_copy(
          out_bufs[b], out_hbm.at[0, :, pl.ds(i0, rows_per_w)], s_sems[b]).wait()

    def transpose_half(b, half):
      @plsc.parallel_loop(0, nl, unroll=8)
      def _t(s):
        rot = lax.bitwise_and(lane_iota + s, nl - 1)
        for ib in range(rows_per_w // nl):
          iv = rot + ib * nl
          for kb in range(half * 2, half * 2 + 2):
            kv = lane_iota + kb * nl
            v = plsc.load_gather(in_bufs[b], [iv, kv])
            plsc.store_scatter(out_bufs[b], [kv, iv], v)

    def transpose_scale(b):
      transpose_half(b, 0)
      transpose_half(b, 1)

    for b in range(NBUF):
      fire_gather(b, b)

    @pl.loop(0, ngroup)
    def _group(g):
      for b in range(NBUF):
        j = g * NBUF + b
        wait_gather(b)

        @pl.when(j >= NBUF)
        def _():
          wait_scatter(b)

        for half in (0, 1):
          transpose_half(b, half)
          fire_scatter_half(j, b, half)

        @pl.when(j + NBUF < cols)
        def _():
          fire_gather(j + NBUF, b)

    for r in range(cols - ngroup * NBUF):
      j = ngroup * NBUF + r
      wait_gather(r)
      wait_scatter(r)
      transpose_scale(r)
      fire_scatter(j, r)

    for b in range(NBUF):
      wait_scatter(b)

  return k(tokens_flat, W_padded)


def kernel(tokens, W):
  rows, cols = tokens.shape
  tokens_flat = tokens.astype(jnp.int32).reshape(-1)
  W_padded = jnp.pad(W * SCALE, ((0, 0), (0, D_PAD - D_MODEL)))
  out_t = _embed(tokens_flat, W_padded, rows=rows, cols=cols)
  return jnp.transpose(out_t, (2, 0, 1))
